# baseline (speedup 1.0000x reference)
import functools

import jax
import jax.numpy as jnp
import numpy as np
from jax import lax
from jax.experimental import pallas as pl
from jax.experimental.pallas import tpu as pltpu
from jax.experimental.pallas import tpu_sc as plsc

_N = 10000
_E = 320000
_NC = 2
_NS = 16
_NW = _NC * _NS
_C = 125
_K = (_E // _NW) // _C
_NBUFS = {16: 10, 32: 10, 48: 10, 80: 5}
_RPT = 624
_REM = _N - _NS * _RPT
_BN = 10000
_ZR = 48


def _gelu(z):
    return z * 0.5 * (1.0 + lax.erf(z * np.float32(1.0 / np.sqrt(2.0))))


@functools.lru_cache(maxsize=None)
def _make_agg(dpad):
    mesh = plsc.VectorSubcoreMesh(core_axis_name="c", subcore_axis_name="s")
    nbuf = _NBUFS[dpad]

    @functools.partial(
        pl.kernel,
        mesh=mesh,
        out_type=jax.ShapeDtypeStruct((_NC, _N, dpad), jnp.float32),
        scratch_types=[
            pltpu.VMEM((_K, _C), jnp.int32),
            pltpu.VMEM((_K, _C), jnp.int32),
            [pltpu.VMEM((_C, dpad), jnp.float32) for _ in range(nbuf)],
            pltpu.VMEM((_ZR, dpad), jnp.float32),
            pltpu.VMEM_SHARED((_N, dpad), jnp.float32),
            [pltpu.SemaphoreType.DMA for _ in range(nbuf)],
            [pltpu.SemaphoreType.DMA for _ in range(nbuf)],
        ],
        compiler_params=pltpu.CompilerParams(use_tc_tiling_on_sc=False),
    )
    def agg(h_hbm, src_hbm, dst_hbm, out_hbm,
            srcv, dstv, bufs, zbuf, acc, gsems, ssems):
        cid = lax.axis_index("c")
        sid = lax.axis_index("s")
        w = sid * _NC + cid

        def zfill(r, _):
            for c in range(dpad // 16):
                zbuf[r, pl.ds(c * 16, 16)] = jnp.zeros((16,), jnp.float32)
            return 0

        lax.fori_loop(0, _ZR, zfill, 0)
        for t in range(_RPT // _ZR):
            pltpu.sync_copy(zbuf, acc.at[pl.ds(sid * _RPT + t * _ZR, _ZR)])

        @pl.when(sid == _NS - 1)
        def _():
            pltpu.sync_copy(zbuf.at[pl.ds(0, _REM)],
                            acc.at[pl.ds(_NS * _RPT, _REM)])
        pltpu.sync_copy(src_hbm.at[w], srcv)
        pltpu.sync_copy(dst_hbm.at[w], dstv)
        plsc.subcore_barrier()

        def gather_start(j, b):
            pltpu.async_copy(h_hbm.at[srcv.at[j]], bufs[b], gsems[b])

        def gather_wait(b):
            pltpu.make_async_copy(h_hbm.at[srcv.at[0]], bufs[b], gsems[b]).wait()

        def scatter_start(j, b):
            pltpu.async_copy(bufs[b], acc.at[dstv.at[j]], ssems[b], add=True)

        def scatter_wait(b):
            pltpu.make_async_copy(bufs[b], acc.at[dstv.at[0]], ssems[b]).wait()

        for b in range(nbuf):
            gather_start(b, b)

        def body(jp, _):
            for b in range(nbuf):
                gather_wait(b)
                scatter_start(nbuf * jp + b, b)
            for b in range(nbuf):
                scatter_wait(b)
                gather_start(nbuf * (jp + 1) + b, b)
            return 0

        lax.fori_loop(0, _K // nbuf - 1, body, 0)
        for b in range(nbuf):
            gather_wait(b)
            scatter_start(_K - nbuf + b, b)
        for b in range(nbuf):
            scatter_wait(b)

        plsc.subcore_barrier()
        pltpu.sync_copy(acc.at[pl.ds(sid * _RPT, _RPT)],
                        out_hbm.at[cid, pl.ds(sid * _RPT, _RPT)])

        @pl.when(sid == _NS - 1)
        def _():
            pltpu.sync_copy(acc.at[pl.ds(_NS * _RPT, _REM)],
                            out_hbm.at[cid, pl.ds(_NS * _RPT, _REM)])

    return agg


def _stage0(x, w, b):
    dop = w.shape[0]

    def body(x_ref, w_ref, b_ref, o_ref):
        z = lax.dot_general(x_ref[...], w_ref[...], (((1,), (1,)), ((), ())),
                            preferred_element_type=jnp.float32)
        o_ref[...] = _gelu(z + b_ref[...])

    return pl.pallas_call(
        body,
        grid=(_N // _BN,),
        in_specs=[
            pl.BlockSpec((_BN, x.shape[1]), lambda i: (i, 0)),
            pl.BlockSpec(w.shape, lambda i: (0, 0)),
            pl.BlockSpec(b.shape, lambda i: (0, 0)),
        ],
        out_specs=pl.BlockSpec((_BN, dop), lambda i: (i, 0)),
        out_shape=jax.ShapeDtypeStruct((_N, dop), jnp.float32),
    )(x, w, b)


def _layer(p, h, w_rel, w_root, b):
    dip = h.shape[1]
    dop = w_rel.shape[0]

    def body(p_ref, h_ref, wr_ref, wo_ref, b_ref, o_ref):
        agg = p_ref[0] + p_ref[1]
        z = lax.dot_general(agg, wr_ref[...], (((1,), (1,)), ((), ())),
                            preferred_element_type=jnp.float32)
        z = z + lax.dot_general(h_ref[...], wo_ref[...], (((1,), (1,)), ((), ())),
                                preferred_element_type=jnp.float32)
        o_ref[...] = _gelu(z + b_ref[...])

    return pl.pallas_call(
        body,
        grid=(_N // _BN,),
        in_specs=[
            pl.BlockSpec((2, _BN, dip), lambda i: (0, i, 0)),
            pl.BlockSpec((_BN, dip), lambda i: (i, 0)),
            pl.BlockSpec(w_rel.shape, lambda i: (0, 0)),
            pl.BlockSpec(w_root.shape, lambda i: (0, 0)),
            pl.BlockSpec(b.shape, lambda i: (0, 0)),
        ],
        out_specs=pl.BlockSpec((_BN, dop), lambda i: (i, 0)),
        out_shape=jax.ShapeDtypeStruct((_N, dop), jnp.float32),
    )(p, h, w_rel, w_root, b)


def _layer_final(p, h, w_rel, w_root, b, wmat):
    dip = h.shape[1]
    dop = w_rel.shape[0]

    def body(p_ref, h_ref, wr_ref, wo_ref, b_ref, wm_ref, o_ref):
        i = pl.program_id(0)
        agg = p_ref[0] + p_ref[1]
        z = lax.dot_general(agg, wr_ref[...], (((1,), (1,)), ((), ())),
                            preferred_element_type=jnp.float32)
        z = z + lax.dot_general(h_ref[...], wo_ref[...], (((1,), (1,)), ((), ())),
                                preferred_element_type=jnp.float32)
        z = _gelu(z + b_ref[...])
        s = jnp.sum(z * wm_ref[...]).reshape(1, 1)

        @pl.when(i == 0)
        def _():
            o_ref[...] = jnp.zeros_like(s)

        o_ref[...] += s

    return pl.pallas_call(
        body,
        grid=(_N // _BN,),
        in_specs=[
            pl.BlockSpec((2, _BN, dip), lambda i: (0, i, 0)),
            pl.BlockSpec((_BN, dip), lambda i: (i, 0)),
            pl.BlockSpec(w_rel.shape, lambda i: (0, 0)),
            pl.BlockSpec(w_root.shape, lambda i: (0, 0)),
            pl.BlockSpec(b.shape, lambda i: (0, 0)),
            pl.BlockSpec((_BN, dop), lambda i: (i, 0)),
        ],
        out_specs=pl.BlockSpec((1, 1), lambda i: (0, 0)),
        out_shape=jax.ShapeDtypeStruct((1, 1), jnp.float32),
    )(p, h, w_rel, w_root, b, wmat)


def _padw(w, dop, dip):
    return jnp.zeros((dop, dip), jnp.float32).at[:w.shape[0], :w.shape[1]].set(w)


def _padb(b, dop):
    return jnp.zeros((1, dop), jnp.float32).at[0, :b.shape[0]].set(b)


def kernel(x, edge_index, w_rel0, b0, w_root0, w_rel1, b1, w_root1,
           w_rel2, b2, w_root2, w_rel3, b3, w_root3, w_rel4, b4, w_root4,
           w_out, b_out):
    src = edge_index[0].reshape(_NW, _K, _C)
    dst = edge_index[1].reshape(_NW, _K, _C)

    h = _stage0(x, _padw(w_root0, 16, 128), _padb(b0, 16))

    layers = [
        (16, 32, w_rel1, w_root1, b1),
        (32, 48, w_rel2, w_root2, b2),
        (48, 80, w_rel3, w_root3, b3),
    ]
    for dip, dop, wr, wo, b in layers:
        p = _make_agg(dip)(h, src, dst)
        h = _layer(p, h, _padw(wr, dop, dip), _padw(wo, dop, dip), _padb(b, dop))

    p = _make_agg(80)(h, src, dst)
    s = _layer_final(p, h, _padw(w_rel4, 160, 80), _padw(w_root4, 160, 80),
                     _padb(b4, 160), w_out.reshape(_N, 160))
    return s.reshape(1) + b_out


# speedup vs baseline: 1.0179x; 1.0179x over previous
import functools

import jax
import jax.numpy as jnp
import numpy as np
from jax import lax
from jax.experimental import pallas as pl
from jax.experimental.pallas import tpu as pltpu
from jax.experimental.pallas import tpu_sc as plsc

_N = 10000
_E = 320000
_NC = 2
_NS = 16
_NW = _NC * _NS
_C = 125
_K = (_E // _NW) // _C
_NBUFS = {16: 10, 32: 10, 48: 10, 80: 5}
_RPT = 624
_REM = _N - _NS * _RPT
_BN = 5000
_ZR = 48


def _gelu(z):
    return z * 0.5 * (1.0 + lax.erf(z * np.float32(1.0 / np.sqrt(2.0))))


@functools.lru_cache(maxsize=None)
def _make_agg(dpad):
    mesh = plsc.VectorSubcoreMesh(core_axis_name="c", subcore_axis_name="s")
    nbuf = _NBUFS[dpad]

    @functools.partial(
        pl.kernel,
        mesh=mesh,
        out_type=jax.ShapeDtypeStruct((_NC, _N, dpad), jnp.float32),
        scratch_types=[
            pltpu.VMEM((_K, _C), jnp.int32),
            pltpu.VMEM((_K, _C), jnp.int32),
            [pltpu.VMEM((_C, dpad), jnp.float32) for _ in range(nbuf)],
            pltpu.VMEM((_ZR, dpad), jnp.float32),
            pltpu.VMEM_SHARED((_N, dpad), jnp.float32),
            [pltpu.SemaphoreType.DMA for _ in range(nbuf)],
            [pltpu.SemaphoreType.DMA for _ in range(nbuf)],
        ],
        compiler_params=pltpu.CompilerParams(use_tc_tiling_on_sc=False),
    )
    def agg(h_hbm, src_hbm, dst_hbm, out_hbm,
            srcv, dstv, bufs, zbuf, acc, gsems, ssems):
        cid = lax.axis_index("c")
        sid = lax.axis_index("s")
        w = sid * _NC + cid

        def zfill(r, _):
            for c in range(dpad // 16):
                zbuf[r, pl.ds(c * 16, 16)] = jnp.zeros((16,), jnp.float32)
            return 0

        lax.fori_loop(0, _ZR, zfill, 0)
        for t in range(_RPT // _ZR):
            pltpu.sync_copy(zbuf, acc.at[pl.ds(sid * _RPT + t * _ZR, _ZR)])

        @pl.when(sid == _NS - 1)
        def _():
            pltpu.sync_copy(zbuf.at[pl.ds(0, _REM)],
                            acc.at[pl.ds(_NS * _RPT, _REM)])
        pltpu.sync_copy(src_hbm.at[w], srcv)
        pltpu.sync_copy(dst_hbm.at[w], dstv)
        plsc.subcore_barrier()

        def gather_start(j, b):
            pltpu.async_copy(h_hbm.at[srcv.at[j]], bufs[b], gsems[b])

        def gather_wait(b):
            pltpu.make_async_copy(h_hbm.at[srcv.at[0]], bufs[b], gsems[b]).wait()

        def scatter_start(j, b):
            pltpu.async_copy(bufs[b], acc.at[dstv.at[j]], ssems[b], add=True)

        def scatter_wait(b):
            pltpu.make_async_copy(bufs[b], acc.at[dstv.at[0]], ssems[b]).wait()

        for b in range(nbuf):
            gather_start(b, b)

        def body(jp, _):
            for b in range(nbuf):
                gather_wait(b)
                scatter_start(nbuf * jp + b, b)
            for b in range(nbuf):
                scatter_wait(b)
                gather_start(nbuf * (jp + 1) + b, b)
            return 0

        lax.fori_loop(0, _K // nbuf - 1, body, 0)
        for b in range(nbuf):
            gather_wait(b)
            scatter_start(_K - nbuf + b, b)
        for b in range(nbuf):
            scatter_wait(b)

        plsc.subcore_barrier()
        pltpu.sync_copy(acc.at[pl.ds(sid * _RPT, _RPT)],
                        out_hbm.at[cid, pl.ds(sid * _RPT, _RPT)])

        @pl.when(sid == _NS - 1)
        def _():
            pltpu.sync_copy(acc.at[pl.ds(_NS * _RPT, _REM)],
                            out_hbm.at[cid, pl.ds(_NS * _RPT, _REM)])

    return agg


def _stage0(x, w, b):
    dop = w.shape[0]

    def body(x_ref, w_ref, b_ref, o_ref):
        z = lax.dot_general(x_ref[...], w_ref[...], (((1,), (1,)), ((), ())),
                            preferred_element_type=jnp.float32)
        o_ref[...] = _gelu(z + b_ref[...])

    return pl.pallas_call(
        body,
        grid=(_N // _BN,),
        in_specs=[
            pl.BlockSpec((_BN, x.shape[1]), lambda i: (i, 0)),
            pl.BlockSpec(w.shape, lambda i: (0, 0)),
            pl.BlockSpec(b.shape, lambda i: (0, 0)),
        ],
        out_specs=pl.BlockSpec((_BN, dop), lambda i: (i, 0)),
        out_shape=jax.ShapeDtypeStruct((_N, dop), jnp.float32),
    )(x, w, b)


def _layer(p, h, w_rel, w_root, b):
    dip = h.shape[1]
    dop = w_rel.shape[0]

    def body(p_ref, h_ref, wr_ref, wo_ref, b_ref, o_ref):
        agg = p_ref[0] + p_ref[1]
        z = lax.dot_general(agg, wr_ref[...], (((1,), (1,)), ((), ())),
                            preferred_element_type=jnp.float32)
        z = z + lax.dot_general(h_ref[...], wo_ref[...], (((1,), (1,)), ((), ())),
                                preferred_element_type=jnp.float32)
        o_ref[...] = _gelu(z + b_ref[...])

    return pl.pallas_call(
        body,
        grid=(_N // _BN,),
        in_specs=[
            pl.BlockSpec((2, _BN, dip), lambda i: (0, i, 0)),
            pl.BlockSpec((_BN, dip), lambda i: (i, 0)),
            pl.BlockSpec(w_rel.shape, lambda i: (0, 0)),
            pl.BlockSpec(w_root.shape, lambda i: (0, 0)),
            pl.BlockSpec(b.shape, lambda i: (0, 0)),
        ],
        out_specs=pl.BlockSpec((_BN, dop), lambda i: (i, 0)),
        out_shape=jax.ShapeDtypeStruct((_N, dop), jnp.float32),
    )(p, h, w_rel, w_root, b)


def _layer_final(p, h, w_rel, w_root, b, wmat):
    dip = h.shape[1]
    dop = w_rel.shape[0]

    def body(p_ref, h_ref, wr_ref, wo_ref, b_ref, wm_ref, o_ref):
        i = pl.program_id(0)
        agg = p_ref[0] + p_ref[1]
        z = lax.dot_general(agg, wr_ref[...], (((1,), (1,)), ((), ())),
                            preferred_element_type=jnp.float32)
        z = z + lax.dot_general(h_ref[...], wo_ref[...], (((1,), (1,)), ((), ())),
                                preferred_element_type=jnp.float32)
        z = _gelu(z + b_ref[...])
        s = jnp.sum(z * wm_ref[...]).reshape(1, 1)

        @pl.when(i == 0)
        def _():
            o_ref[...] = jnp.zeros_like(s)

        o_ref[...] += s

    return pl.pallas_call(
        body,
        grid=(_N // _BN,),
        in_specs=[
            pl.BlockSpec((2, _BN, dip), lambda i: (0, i, 0)),
            pl.BlockSpec((_BN, dip), lambda i: (i, 0)),
            pl.BlockSpec(w_rel.shape, lambda i: (0, 0)),
            pl.BlockSpec(w_root.shape, lambda i: (0, 0)),
            pl.BlockSpec(b.shape, lambda i: (0, 0)),
            pl.BlockSpec((_BN, dop), lambda i: (i, 0)),
        ],
        out_specs=pl.BlockSpec((1, 1), lambda i: (0, 0)),
        out_shape=jax.ShapeDtypeStruct((1, 1), jnp.float32),
    )(p, h, w_rel, w_root, b, wmat)


def _padw(w, dop, dip):
    return jnp.zeros((dop, dip), jnp.float32).at[:w.shape[0], :w.shape[1]].set(w)


def _padb(b, dop):
    return jnp.zeros((1, dop), jnp.float32).at[0, :b.shape[0]].set(b)


def kernel(x, edge_index, w_rel0, b0, w_root0, w_rel1, b1, w_root1,
           w_rel2, b2, w_root2, w_rel3, b3, w_root3, w_rel4, b4, w_root4,
           w_out, b_out):
    src = edge_index[0].reshape(_NW, _K, _C)
    dst = edge_index[1].reshape(_NW, _K, _C)

    h = _stage0(x, _padw(w_root0, 16, 128), _padb(b0, 16))

    layers = [
        (16, 32, w_rel1, w_root1, b1),
        (32, 48, w_rel2, w_root2, b2),
        (48, 80, w_rel3, w_root3, b3),
    ]
    for dip, dop, wr, wo, b in layers:
        p = _make_agg(dip)(h, src, dst)
        h = _layer(p, h, _padw(wr, dop, dip), _padw(wo, dop, dip), _padb(b, dop))

    p = _make_agg(80)(h, src, dst)
    s = _layer_final(p, h, _padw(w_rel4, 160, 80), _padw(w_root4, 160, 80),
                     _padb(b4, 160), w_out.reshape(_N, 160))
    return s.reshape(1) + b_out


# speedup vs baseline: 1.1223x; 1.1026x over previous
import functools

import jax
import jax.numpy as jnp
import numpy as np
from jax import lax
from jax.experimental import pallas as pl
from jax.experimental.pallas import tpu as pltpu
from jax.experimental.pallas import tpu_sc as plsc

_N = 10000
_E = 320000
_NC = 2
_NS = 16
_NW = _NC * _NS
_C = 125
_K = (_E // _NW) // _C
_NBUFS = {16: 10, 32: 10, 48: 10, 80: 5}
_RPT = 624
_REM = _N - _NS * _RPT
_BN = 5000
_ZR = 48


def _gelu(z):
    return z * 0.5 * (1.0 + lax.erf(z * np.float32(1.0 / np.sqrt(2.0))))


@functools.lru_cache(maxsize=None)
def _make_agg(dpad):
    mesh = plsc.VectorSubcoreMesh(core_axis_name="c", subcore_axis_name="s")
    nbuf = _NBUFS[dpad]

    @functools.partial(
        pl.kernel,
        mesh=mesh,
        out_type=jax.ShapeDtypeStruct((_NC, _N, 128), jnp.float32),
        scratch_types=[
            pltpu.VMEM((_K, _C), jnp.int32),
            pltpu.VMEM((_K, _C), jnp.int32),
            [pltpu.VMEM((_C, dpad), jnp.float32) for _ in range(nbuf)],
            pltpu.VMEM((_ZR, dpad), jnp.float32),
            pltpu.VMEM_SHARED((_N, dpad), jnp.float32),
            [pltpu.SemaphoreType.DMA for _ in range(nbuf)],
            [pltpu.SemaphoreType.DMA for _ in range(nbuf)],
        ],
        compiler_params=pltpu.CompilerParams(use_tc_tiling_on_sc=False),
    )
    def agg(h_hbm, src_hbm, dst_hbm, out_hbm,
            srcv, dstv, bufs, zbuf, acc, gsems, ssems):
        cid = lax.axis_index("c")
        sid = lax.axis_index("s")
        w = sid * _NC + cid

        def zfill(r, _):
            for c in range(dpad // 16):
                zbuf[r, pl.ds(c * 16, 16)] = jnp.zeros((16,), jnp.float32)
            return 0

        lax.fori_loop(0, _ZR, zfill, 0)
        for t in range(_RPT // _ZR):
            pltpu.sync_copy(zbuf, acc.at[pl.ds(sid * _RPT + t * _ZR, _ZR)])

        @pl.when(sid == _NS - 1)
        def _():
            pltpu.sync_copy(zbuf.at[pl.ds(0, _REM)],
                            acc.at[pl.ds(_NS * _RPT, _REM)])
        pltpu.sync_copy(src_hbm.at[w], srcv)
        pltpu.sync_copy(dst_hbm.at[w], dstv)
        plsc.subcore_barrier()

        def gather_start(j, b):
            pltpu.async_copy(h_hbm.at[srcv.at[j]], bufs[b], gsems[b])

        def gather_wait(b):
            pltpu.make_async_copy(h_hbm.at[srcv.at[0]], bufs[b], gsems[b]).wait()

        def scatter_start(j, b):
            pltpu.async_copy(bufs[b], acc.at[dstv.at[j]], ssems[b], add=True)

        def scatter_wait(b):
            pltpu.make_async_copy(bufs[b], acc.at[dstv.at[0]], ssems[b]).wait()

        for b in range(nbuf):
            gather_start(b, b)

        def body(jp, _):
            for b in range(nbuf):
                gather_wait(b)
                scatter_start(nbuf * jp + b, b)
            for b in range(nbuf):
                scatter_wait(b)
                gather_start(nbuf * (jp + 1) + b, b)
            return 0

        lax.fori_loop(0, _K // nbuf - 1, body, 0)
        for b in range(nbuf):
            gather_wait(b)
            scatter_start(_K - nbuf + b, b)
        for b in range(nbuf):
            scatter_wait(b)

        plsc.subcore_barrier()
        pltpu.sync_copy(acc.at[pl.ds(sid * _RPT, _RPT)],
                        out_hbm.at[cid, pl.ds(sid * _RPT, _RPT), pl.ds(0, dpad)])

        @pl.when(sid == _NS - 1)
        def _():
            pltpu.sync_copy(acc.at[pl.ds(_NS * _RPT, _REM)],
                            out_hbm.at[cid, pl.ds(_NS * _RPT, _REM), pl.ds(0, dpad)])

    return agg


def _stage0(x, w, b):
    dop = w.shape[0]

    def body(x_ref, w_ref, b_ref, o_ref):
        z = lax.dot_general(x_ref[...], w_ref[...], (((1,), (1,)), ((), ())),
                            preferred_element_type=jnp.float32)
        o_ref[...] = _gelu(z + b_ref[...])

    return pl.pallas_call(
        body,
        grid=(_N // _BN,),
        in_specs=[
            pl.BlockSpec((_BN, x.shape[1]), lambda i: (i, 0)),
            pl.BlockSpec(w.shape, lambda i: (0, 0)),
            pl.BlockSpec(b.shape, lambda i: (0, 0)),
        ],
        out_specs=pl.BlockSpec((_BN, dop), lambda i: (i, 0)),
        out_shape=jax.ShapeDtypeStruct((_N, dop), jnp.float32),
    )(x, w, b)


def _layer(p, h, w_rel, w_root, b):
    dop, dip = w_rel.shape

    def body(p_ref, h_ref, wr_ref, wo_ref, b_ref, o_ref):
        agg = (p_ref[0] + p_ref[1])[:, :dip]
        z = lax.dot_general(agg, wr_ref[...], (((1,), (1,)), ((), ())),
                            preferred_element_type=jnp.float32)
        z = z + lax.dot_general(h_ref[...], wo_ref[...], (((1,), (1,)), ((), ())),
                                preferred_element_type=jnp.float32)
        o_ref[...] = _gelu(z + b_ref[...])

    return pl.pallas_call(
        body,
        grid=(_N // _BN,),
        in_specs=[
            pl.BlockSpec((2, _BN, 128), lambda i: (0, i, 0)),
            pl.BlockSpec((_BN, dip), lambda i: (i, 0)),
            pl.BlockSpec(w_rel.shape, lambda i: (0, 0)),
            pl.BlockSpec(w_root.shape, lambda i: (0, 0)),
            pl.BlockSpec(b.shape, lambda i: (0, 0)),
        ],
        out_specs=pl.BlockSpec((_BN, dop), lambda i: (i, 0)),
        out_shape=jax.ShapeDtypeStruct((_N, dop), jnp.float32),
    )(p, h, w_rel, w_root, b)


def _layer_final(p, h, w_rel, w_root, b, wmat):
    dop, dip = w_rel.shape

    def body(p_ref, h_ref, wr_ref, wo_ref, b_ref, wm_ref, o_ref):
        i = pl.program_id(0)
        agg = (p_ref[0] + p_ref[1])[:, :dip]
        z = lax.dot_general(agg, wr_ref[...], (((1,), (1,)), ((), ())),
                            preferred_element_type=jnp.float32)
        z = z + lax.dot_general(h_ref[...], wo_ref[...], (((1,), (1,)), ((), ())),
                                preferred_element_type=jnp.float32)
        z = _gelu(z + b_ref[...])
        s = jnp.sum(z * wm_ref[...]).reshape(1, 1)

        @pl.when(i == 0)
        def _():
            o_ref[...] = jnp.zeros_like(s)

        o_ref[...] += s

    return pl.pallas_call(
        body,
        grid=(_N // _BN,),
        in_specs=[
            pl.BlockSpec((2, _BN, 128), lambda i: (0, i, 0)),
            pl.BlockSpec((_BN, dip), lambda i: (i, 0)),
            pl.BlockSpec(w_rel.shape, lambda i: (0, 0)),
            pl.BlockSpec(w_root.shape, lambda i: (0, 0)),
            pl.BlockSpec(b.shape, lambda i: (0, 0)),
            pl.BlockSpec((_BN, dop), lambda i: (i, 0)),
        ],
        out_specs=pl.BlockSpec((1, 1), lambda i: (0, 0)),
        out_shape=jax.ShapeDtypeStruct((1, 1), jnp.float32),
    )(p, h, w_rel, w_root, b, wmat)


def _padw(w, dop, dip):
    return jnp.zeros((dop, dip), jnp.float32).at[:w.shape[0], :w.shape[1]].set(w)


def _padb(b, dop):
    return jnp.zeros((1, dop), jnp.float32).at[0, :b.shape[0]].set(b)


def kernel(x, edge_index, w_rel0, b0, w_root0, w_rel1, b1, w_root1,
           w_rel2, b2, w_root2, w_rel3, b3, w_root3, w_rel4, b4, w_root4,
           w_out, b_out):
    src = edge_index[0].reshape(_NW, _K, _C)
    dst = edge_index[1].reshape(_NW, _K, _C)

    h = _stage0(x, _padw(w_root0, 16, 128), _padb(b0, 16))

    layers = [
        (16, 32, w_rel1, w_root1, b1),
        (32, 48, w_rel2, w_root2, b2),
        (48, 80, w_rel3, w_root3, b3),
    ]
    for dip, dop, wr, wo, b in layers:
        p = _make_agg(dip)(h, src, dst)
        h = _layer(p, h, _padw(wr, dop, dip), _padw(wo, dop, dip), _padb(b, dop))

    p = _make_agg(80)(h, src, dst)
    s = _layer_final(p, h, _padw(w_rel4, 160, 80), _padw(w_root4, 160, 80),
                     _padb(b4, 160), w_out.reshape(_N, 160))
    return s.reshape(1) + b_out
